# Initial kernel scaffold; baseline (speedup 1.0000x reference)
import jax
import jax.numpy as jnp
from jax.experimental import pallas as pl


def kernel(x, y):
    raise NotImplementedError("write your pallas kernel here")


# speedup vs baseline: 3.3344x; 3.3344x over previous
import functools

import jax
import jax.numpy as jnp
from jax.experimental import pallas as pl
from jax.experimental.pallas import tpu as pltpu

_K = 16
_NPTS_DS = 1024
_ROWS = 256


def _knn_body(xb_ref, xr_ref, idx_ref):
    xb = xb_ref[0]
    xr = xr_ref[0]
    n = xb.shape[1]
    r = xr.shape[1]

    sq_c = (xb[0] * xb[0] + xb[1] * xb[1]) + xb[2] * xb[2]
    sq_r = (xr[0] * xr[0] + xr[1] * xr[1]) + xr[2] * xr[2]

    inner = jax.lax.dot_general(
        xr, xb, (((0,), (0,)), ((), ())),
        preferred_element_type=jnp.float32)

    neg = (2.0 * inner - sq_r[:, None]) - sq_c[None, :]

    iota = jax.lax.broadcasted_iota(jnp.int32, (r, n), 1)
    cols = []
    for _ in range(_K):
        m = jnp.max(neg, axis=1, keepdims=True)
        cand = jnp.where(neg == m, iota, n)
        j = jnp.min(cand, axis=1, keepdims=True)
        cols.append(j)
        neg = jnp.where(iota == j, -jnp.inf, neg)
    idx_ref[0] = jnp.concatenate(cols, axis=1)


def _knn_idx(x):
    b, _, n = x.shape
    grid = (b, n // _ROWS)
    return pl.pallas_call(
        _knn_body,
        grid=grid,
        in_specs=[
            pl.BlockSpec((1, 3, n), lambda bi, ri: (bi, 0, 0)),
            pl.BlockSpec((1, 3, _ROWS), lambda bi, ri: (bi, 0, ri)),
        ],
        out_specs=pl.BlockSpec((1, _ROWS, _K), lambda bi, ri: (bi, ri, 0)),
        out_shape=jax.ShapeDtypeStruct((b, n, _K), jnp.int32),
    )(x, x)


def _gather_channels(t, idx):
    return jax.vmap(lambda tb, ib: tb[:, ib])(t, idx)


def kernel(x, y):
    idx_knn = _knn_idx(x)
    neighbor = _gather_channels(x, idx_knn)
    value = jnp.std(neighbor, axis=-1, ddof=1)
    score = jnp.sum(value, axis=1)
    _, idx = jax.lax.top_k(score, _NPTS_DS)
    top_k_xyz = _gather_channels(x, idx)
    top_k_points = _gather_channels(y, idx)
    return (top_k_xyz, top_k_points)


# speedup vs baseline: 11.4818x; 3.4434x over previous
import functools

import jax
import jax.numpy as jnp
from jax.experimental import pallas as pl
from jax.experimental.pallas import tpu as pltpu

_K = 16
_NPTS_DS = 1024
_ROWS = 256


def _tree16(vals):
    s = [vals[t] + vals[t + 8] for t in range(8)]
    u = [s[t] + s[t + 4] for t in range(4)]
    w = [u[t] + u[t + 2] for t in range(2)]
    return w[0] + w[1]


def _score_body(xb_ref, xr_ref, xt_ref, score_ref):
    xb = xb_ref[0]
    xr = xr_ref[0]
    xt = xt_ref[0]
    n = xb.shape[1]
    r = xr.shape[1]

    sq_c = (xb[0] * xb[0] + xb[1] * xb[1]) + xb[2] * xb[2]
    sq_r = (xr[0] * xr[0] + xr[1] * xr[1]) + xr[2] * xr[2]

    inner = jax.lax.dot_general(
        xr, xb, (((0,), (0,)), ((), ())),
        preferred_element_type=jnp.float32)

    neg = (2.0 * inner - sq_r[:, None]) - sq_c[None, :]

    a16 = xt.astype(jnp.bfloat16)
    r1 = xt - a16.astype(jnp.float32)
    b16 = r1.astype(jnp.bfloat16)
    r2 = r1 - b16.astype(jnp.float32)
    c16 = r2.astype(jnp.bfloat16)
    abc = jnp.concatenate([a16, b16, c16], axis=1)

    iota = jax.lax.broadcasted_iota(jnp.int32, (r, n), 1)
    vals = []
    for _ in range(_K):
        m = jnp.max(neg, axis=1, keepdims=True)
        cand = jnp.where(neg == m, iota, n)
        j = jnp.min(cand, axis=1, keepdims=True)
        sel = iota == j
        neg = jnp.where(sel, -jnp.inf, neg)
        onehot = jnp.where(sel, 1.0, 0.0).astype(jnp.bfloat16)
        picked = jax.lax.dot_general(
            onehot, abc, (((1,), (0,)), ((), ())),
            preferred_element_type=jnp.float32)
        vals.append((picked[:, 0:3] + picked[:, 3:6]) + picked[:, 6:9])

    mean = _tree16(vals) * jnp.float32(0.0625)
    sqs = [(v - mean) * (v - mean) for v in vals]
    var = _tree16(sqs) * jnp.float32(1.0 / 15.0)
    std = jnp.sqrt(var)
    score = (std[:, 0] + std[:, 2]) + std[:, 1]
    score_ref[0, 0] = score


def _point_scores(x):
    b, _, n = x.shape
    xt = jnp.swapaxes(x, 1, 2)
    nb = n // _ROWS
    grid = (b, nb)
    out = pl.pallas_call(
        _score_body,
        grid=grid,
        in_specs=[
            pl.BlockSpec((1, 3, n), lambda bi, ri: (bi, 0, 0)),
            pl.BlockSpec((1, 3, _ROWS), lambda bi, ri: (bi, 0, ri)),
            pl.BlockSpec((1, n, 3), lambda bi, ri: (bi, 0, 0)),
        ],
        out_specs=pl.BlockSpec(
            (1, 1, _ROWS), lambda bi, ri: (bi * nb + ri, 0, 0)),
        out_shape=jax.ShapeDtypeStruct((b * nb, 1, _ROWS), jnp.float32),
    )(x, x, xt)
    return out.reshape(b, n)


def _gather_channels(t, idx):
    return jax.vmap(lambda tb, ib: tb[:, ib])(t, idx)


def kernel(x, y):
    score = _point_scores(x)
    _, idx = jax.lax.top_k(score, _NPTS_DS)
    top_k_xyz = _gather_channels(x, idx)
    top_k_points = _gather_channels(y, idx)
    return (top_k_xyz, top_k_points)
